# baseline (speedup 1.0000x reference)
import jax
import jax.numpy as jnp
from jax import lax
from jax.experimental import pallas as pl
from jax.experimental.pallas import tpu as pltpu
from jax.experimental.pallas import tpu_sc as plsc


def _sc_mean(context, emb_table, num_cores=2, num_subcores=16):
    B, CTX = context.shape
    V, D = emb_table.shape
    NW = num_cores * num_subcores
    BPW = B // NW
    IPW = BPW * CTX
    CHUNK = 80
    NCH = IPW // CHUNK
    LANES = 16
    DCH = D // LANES

    ctx3 = context.astype(jnp.int32).reshape(NW, NCH, CHUNK)

    def body(ctx_hbm, emb_hbm, out_hbm, idx_v, rows_v, mean_v, sem):
        wid = lax.axis_index("s") * num_cores + lax.axis_index("c")
        pltpu.sync_copy(ctx_hbm.at[wid], idx_v)
        copies = [
            pltpu.async_copy(
                emb_hbm.at[idx_v.at[k]], rows_v.at[pl.ds(k * CHUNK, CHUNK)], sem
            )
            for k in range(NCH)
        ]
        for c in copies:
            c.wait()

        scale = jnp.float32(1.0 / CTX)

        def mean_one(i, _):
            base = i * CTX
            for c in range(DCH):
                acc = rows_v[base, pl.ds(c * LANES, LANES)]
                for j in range(1, CTX):
                    acc = acc + rows_v[base + j, pl.ds(c * LANES, LANES)]
                mean_v[i, pl.ds(c * LANES, LANES)] = acc * scale
            return 0

        lax.fori_loop(0, BPW, mean_one, 0)
        pltpu.sync_copy(mean_v, out_hbm.at[pl.ds(wid * BPW, BPW)])

    mesh = plsc.VectorSubcoreMesh(
        core_axis_name="c", subcore_axis_name="s",
        num_cores=num_cores, num_subcores=num_subcores,
    )
    return pl.kernel(
        body,
        out_type=jax.ShapeDtypeStruct((B, D), jnp.float32),
        mesh=mesh,
        compiler_params=pltpu.CompilerParams(use_tc_tiling_on_sc=False),
        scratch_types=[
            pltpu.VMEM((NCH, CHUNK), jnp.int32),
            pltpu.VMEM((IPW, D), jnp.float32),
            pltpu.VMEM((BPW, D), jnp.float32),
            pltpu.SemaphoreType.DMA,
        ],
    )(ctx3, emb_table)


_NBUF = 2


def _dot_nt(a, b):
    return lax.dot_general(
        a, b, dimension_numbers=(((1,), (1,)), ((), ())),
        preferred_element_type=jnp.float32,
    )


def _tc_logits(mean, W, mb=32):
    B, D = mean.shape
    V, _ = W.shape
    S = 49920
    T = V - 2 * S
    nbands = B // mb

    W_bf = W.astype(jnp.bfloat16)

    def body(mean_ref, wa_ref, wb_ref, wc_ref, out_ref, obuf, sems):
        i = pl.program_id(0)

        def descs(j):
            s = lax.rem(j, _NBUF)
            return [
                pltpu.make_async_copy(
                    obuf.at[s, pl.ds(k * (mb // 2), mb // 2)],
                    out_ref.at[pl.ds(j * mb + k * (mb // 2), mb // 2)],
                    sems.at[s, k])
                for k in range(2)
            ]

        @pl.when(i >= _NBUF)
        def _():
            for d in descs(i - _NBUF):
                d.wait()

        s = lax.rem(i, _NBUF)
        m_bf = mean_ref[...].astype(jnp.bfloat16)
        obuf[s, :, pl.ds(0, S)] = _dot_nt(m_bf, wa_ref[...])
        obuf[s, :, pl.ds(S, S)] = _dot_nt(m_bf, wb_ref[...])
        obuf[s, :, pl.ds(2 * S, T)] = _dot_nt(m_bf, wc_ref[...])
        for d in descs(i):
            d.start()

        @pl.when(i == nbands - 1)
        def _():
            for k in range(_NBUF):
                for d in descs(i - (_NBUF - 1) + k):
                    d.wait()

    return pl.pallas_call(
        body,
        grid=(nbands,),
        in_specs=[
            pl.BlockSpec((mb, D), lambda i: (i, 0)),
            pl.BlockSpec((S, D), lambda i: (0, 0)),
            pl.BlockSpec((S, D), lambda i: (1, 0)),
            pl.BlockSpec((T, D), lambda i: (2 * S // T, 0)),
        ],
        out_specs=pl.BlockSpec(memory_space=pl.ANY),
        out_shape=jax.ShapeDtypeStruct((B, V), jnp.float32),
        scratch_shapes=[
            pltpu.VMEM((_NBUF, mb, V), jnp.float32),
            pltpu.SemaphoreType.DMA((_NBUF, 2)),
        ],
        compiler_params=pltpu.CompilerParams(
            vmem_limit_bytes=62 * 1024 * 1024,
        ),
    )(mean, W_bf, W_bf, W_bf)


def kernel(context, emb_table, W):
    mean = _sc_mean(context, emb_table)
    return _tc_logits(mean, W)


# speedup vs baseline: 3.4811x; 3.4811x over previous
import jax
import jax.numpy as jnp
from jax import lax
from jax.experimental import pallas as pl
from jax.experimental.pallas import tpu as pltpu
from jax.experimental.pallas import tpu_sc as plsc


def _sc_mean(context, emb_table, num_cores=2, num_subcores=16):
    B, CTX = context.shape
    V, D = emb_table.shape
    NW = num_cores * num_subcores
    BPW = B // NW
    IPW = BPW * CTX
    CHUNK = 80
    NCH = IPW // CHUNK
    LANES = 16
    DCH = D // LANES

    ctx3 = context.astype(jnp.int32).reshape(NW, NCH, CHUNK)

    def body(ctx_hbm, emb_hbm, out_hbm, idx_v, rows_v, mean_v, sem):
        wid = lax.axis_index("s") * num_cores + lax.axis_index("c")
        pltpu.sync_copy(ctx_hbm.at[wid], idx_v)
        copies = [
            pltpu.async_copy(
                emb_hbm.at[idx_v.at[k]], rows_v.at[pl.ds(k * CHUNK, CHUNK)], sem
            )
            for k in range(NCH)
        ]
        for c in copies:
            c.wait()

        scale = jnp.float32(1.0 / CTX)

        def mean_one(i, _):
            base = i * CTX
            for c in range(DCH):
                acc = rows_v[base, pl.ds(c * LANES, LANES)]
                for j in range(1, CTX):
                    acc = acc + rows_v[base + j, pl.ds(c * LANES, LANES)]
                mean_v[i, pl.ds(c * LANES, LANES)] = acc * scale
            return 0

        lax.fori_loop(0, BPW, mean_one, 0)
        pltpu.sync_copy(mean_v, out_hbm.at[pl.ds(wid * BPW, BPW)])

    mesh = plsc.VectorSubcoreMesh(
        core_axis_name="c", subcore_axis_name="s",
        num_cores=num_cores, num_subcores=num_subcores,
    )
    return pl.kernel(
        body,
        out_type=jax.ShapeDtypeStruct((B, D), jnp.float32),
        mesh=mesh,
        compiler_params=pltpu.CompilerParams(use_tc_tiling_on_sc=False),
        scratch_types=[
            pltpu.VMEM((NCH, CHUNK), jnp.int32),
            pltpu.VMEM((IPW, D), jnp.float32),
            pltpu.VMEM((BPW, D), jnp.float32),
            pltpu.SemaphoreType.DMA,
        ],
    )(ctx3, emb_table)


def _mmt_body(mean_ref, wt_ref, out_ref):
    out_ref[...] = lax.dot_general(
        wt_ref[...].astype(jnp.bfloat16), mean_ref[...].astype(jnp.bfloat16),
        dimension_numbers=(((0,), (1,)), ((), ())),
        preferred_element_type=jnp.float32,
    )


def _tc_logits_t(mean, W_T, vb=4096):
    B, D = mean.shape
    _, V = W_T.shape
    return pl.pallas_call(
        _mmt_body,
        grid=(pl.cdiv(V, vb),),
        in_specs=[
            pl.BlockSpec((B, D), lambda i: (0, 0)),
            pl.BlockSpec((D, vb), lambda i: (0, i)),
        ],
        out_specs=pl.BlockSpec((vb, B), lambda i: (i, 0)),
        out_shape=jax.ShapeDtypeStruct((V, B), jnp.float32),
        compiler_params=pltpu.CompilerParams(
            vmem_limit_bytes=60 * 1024 * 1024,
        ),
    )(mean, W_T)


def kernel(context, emb_table, W):
    mean = _sc_mean(context, emb_table)
    return _tc_logits_t(mean, W.T).T
